# baseline (speedup 1.0000x reference)
import jax
import jax.numpy as jnp
from jax.experimental import pallas as pl
from jax.experimental.pallas import tpu as pltpu


def _dot(a, b):
    return jnp.dot(a.astype(jnp.bfloat16), b.astype(jnp.bfloat16),
                   preferred_element_type=jnp.float32)


_QW = 24
_NQ = 624
_NY1 = 552
_NY2 = 528
_NB = 8


def _tap_src(q, d):
    t = q + d - 1
    p = t % 2
    s = (t - p) // 2
    return p, s


def _conv_stage_body(xq_ref, w1_ref, b1_ref, w2_ref, b2_ref, out_ref):
    uv = jax.lax.broadcasted_iota(jnp.int32, (1, _NY1), 1)
    for j in range(_NB):
        xq = [xq_ref[j, q * _NQ:(q + 1) * _NQ] for q in range(4)]
        y1m = {}
        for qr in range(2):
            for qc in range(2):
                rows = []
                for di in range(3):
                    pr, a = _tap_src(qr, di)
                    for dj in range(3):
                        pc, b = _tap_src(qc, dj)
                        off = (1 + a) * _QW + (1 + b)
                        rows.append(
                            xq[2 * pr + pc][off:off + _NY1].reshape(1, _NY1))
                cols1 = jnp.concatenate(rows, axis=0)
                y1 = _dot(w1_ref[...], cols1) + b1_ref[...]
                y1 = jnp.maximum(y1, 0.0)
                ok = ((uv % _QW < 23 - qc) & (uv // _QW < 23 - qr))
                y1 = y1 * ok.astype(jnp.float32)
                y1m[(qr, qc)] = jnp.concatenate(
                    [jnp.zeros((32, _QW + 1), jnp.float32), y1,
                     jnp.zeros((32, _NQ - _NY1 - _QW - 1), jnp.float32)],
                    axis=1)
        pooled = None
        for a in range(2):
            for b in range(2):
                pieces = []
                for di in range(3):
                    pr, al = _tap_src(a, di)
                    for dj in range(3):
                        pc, be = _tap_src(b, dj)
                        off = (1 + al) * _QW + (1 + be)
                        pieces.append(y1m[(pr, pc)][:, off:off + _NY2])
                cols2 = jnp.concatenate(pieces, axis=0)
                y2 = _dot(w2_ref[...], cols2) + b2_ref[...]
                pooled = y2 if pooled is None else jnp.maximum(pooled, y2)
        out_ref[:, j, :] = pooled.astype(jnp.bfloat16)


def _fc_stage_body(xp_ref, w3_ref, fc1b_ref, fc2w_ref, fc2b_ref, len_ref,
                   vals_ref, idx_ref, acc_ref):
    c = pl.program_id(0)

    @pl.when(c == 0)
    def _():
        acc_ref[...] = jnp.zeros_like(acc_ref)

    acc_ref[...] += _dot(xp_ref[0], w3_ref[0])

    @pl.when(c == pl.num_programs(0) - 1)
    def _():
        y = jnp.maximum(acc_ref[...] + fc1b_ref[...], 0.0)
        logits = _dot(y, fc2w_ref[...]) + fc2b_ref[...]
        mx = jnp.max(logits, axis=1, keepdims=True)
        e = jnp.exp(logits - mx)
        probs = e / jnp.sum(e, axis=1, keepdims=True)
        row_l = jax.lax.broadcasted_iota(jnp.int32, (448, 1), 0) % 7
        keep = (row_l < len_ref[...]).astype(jnp.float32)
        lanes = jax.lax.broadcasted_iota(jnp.int32, (448, 16), 1)
        vals_ref[...] = jnp.zeros_like(vals_ref)
        idx_ref[...] = jnp.zeros_like(idx_ref)
        work = probs
        for k in range(7):
            top = jnp.max(work, axis=1, keepdims=True)
            sel = jnp.min(jnp.where(work == top, lanes, 9999), axis=1,
                          keepdims=True)
            vals_ref[:, k:k + 1] = top * keep
            idx_ref[:, k:k + 1] = sel
            work = jnp.where(lanes == sel, -jnp.inf, work)


def kernel(img_seq, img_seq_len, conv1_w, conv1_b, conv2_w, conv2_b,
           fc1_w, fc1_b, fc2_w, fc2_b):
    B, L = img_seq.shape[0], img_seq.shape[1]
    N = B * L
    x = img_seq.reshape(N, 45, 45)
    quarters = []
    for pr in range(2):
        for pc in range(2):
            q = x[:, pr::2, pc::2]
            q = jnp.pad(q, ((0, 0), (1, 25 - q.shape[1]), (1, 23 - q.shape[2])))
            quarters.append(q.reshape(N, _NQ))
    xq = jnp.concatenate(quarters, axis=1)

    w1f = conv1_w.reshape(32, 9)
    b1 = conv1_b.reshape(32, 1)
    w2r = conv2_w.transpose(0, 2, 3, 1).reshape(64, 288)
    b2 = conv2_b.reshape(64, 1)

    pooled = pl.pallas_call(
        _conv_stage_body,
        grid=(N // _NB,),
        in_specs=[
            pl.BlockSpec((_NB, 4 * _NQ), lambda i: (i, 0)),
            pl.BlockSpec((32, 9), lambda i: (0, 0)),
            pl.BlockSpec((32, 1), lambda i: (0, 0)),
            pl.BlockSpec((64, 288), lambda i: (0, 0)),
            pl.BlockSpec((64, 1), lambda i: (0, 0)),
        ],
        out_specs=pl.BlockSpec((64, _NB, _NY2), lambda i: (0, i, 0)),
        out_shape=jax.ShapeDtypeStruct((64, N, _NY2), jnp.bfloat16),
    )(xq, w1f, b1, w2r, b2)

    w3 = fc1_w.reshape(128, 64, 22, 22)
    w3 = jnp.pad(w3, ((0, 0), (0, 0), (0, 0), (0, 2)))
    w3 = w3.reshape(128, 64, _NY2).transpose(1, 2, 0).astype(jnp.bfloat16)
    fc1b = fc1_b.reshape(1, 128)
    fc2wt = jnp.zeros((128, 16), jnp.float32).at[:, :14].set(fc2_w.T)
    fc2b = jnp.full((1, 16), -1e30, jnp.float32).at[0, :14].set(fc2_b)
    len_rep = jnp.repeat(img_seq_len.astype(jnp.int32), L).reshape(N, 1)

    vals, idx = pl.pallas_call(
        _fc_stage_body,
        grid=(64,),
        in_specs=[
            pl.BlockSpec((1, N, _NY2), lambda c: (c, 0, 0)),
            pl.BlockSpec((1, _NY2, 128), lambda c: (c, 0, 0)),
            pl.BlockSpec((1, 128), lambda c: (0, 0)),
            pl.BlockSpec((128, 16), lambda c: (0, 0)),
            pl.BlockSpec((1, 16), lambda c: (0, 0)),
            pl.BlockSpec((N, 1), lambda c: (0, 0)),
        ],
        out_specs=[
            pl.BlockSpec((N, 16), lambda c: (0, 0)),
            pl.BlockSpec((N, 16), lambda c: (0, 0)),
        ],
        out_shape=[
            jax.ShapeDtypeStruct((N, 16), jnp.float32),
            jax.ShapeDtypeStruct((N, 16), jnp.int32),
        ],
        scratch_shapes=[pltpu.VMEM((N, 128), jnp.float32)],
    )(pooled, w3, fc1b, fc2wt, fc2b, len_rep)

    top_vals = vals[:, :7].reshape(B, L, 7)
    top_idx = idx[:, :7].reshape(B, L, 7)
    return top_vals, top_idx


# speedup vs baseline: 1.3597x; 1.3597x over previous
import jax
import jax.numpy as jnp
from jax.experimental import pallas as pl
from jax.experimental.pallas import tpu as pltpu


def _dot(a, b):
    return jnp.dot(a.astype(jnp.bfloat16), b.astype(jnp.bfloat16),
                   preferred_element_type=jnp.float32)


_QW = 24
_NQ = 624
_NY1 = 552
_NY2 = 528
_NB = 8


def _tap_src(q, d):
    t = q + d - 1
    p = t % 2
    s = (t - p) // 2
    return p, s


def _conv_stage_body(xq_ref, w1_ref, b1_ref, w2_ref, b2_ref, out_ref):
    uv = jax.lax.broadcasted_iota(jnp.int32, (1, _NY1), 1)
    y1m = {}
    for qr in range(2):
        for qc in range(2):
            taps = []
            for di in range(3):
                pr, a = _tap_src(qr, di)
                for dj in range(3):
                    pc, b = _tap_src(qc, dj)
                    off = (2 * pr + pc) * _NQ + (1 + a) * _QW + (1 + b)
                    taps.append(xq_ref[:, off:off + _NY1])
            cols1 = jnp.concatenate(taps, axis=0)
            y1 = _dot(w1_ref[...], cols1) + b1_ref[...]
            y1 = jnp.maximum(y1, 0.0)
            ok = ((uv % _QW < 23 - qc) & (uv // _QW < 23 - qr))
            y1 = y1 * ok.astype(jnp.float32)
            y1m[(qr, qc)] = jnp.concatenate(
                [jnp.zeros((_NB * 32, _QW + 1), jnp.float32), y1,
                 jnp.zeros((_NB * 32, _NQ - _NY1 - _QW - 1), jnp.float32)],
                axis=1)
    for j in range(_NB):
        pooled = None
        for a in range(2):
            for b in range(2):
                pieces = []
                for di in range(3):
                    pr, al = _tap_src(a, di)
                    for dj in range(3):
                        pc, be = _tap_src(b, dj)
                        off = (1 + al) * _QW + (1 + be)
                        pieces.append(
                            y1m[(pr, pc)][j * 32:(j + 1) * 32,
                                          off:off + _NY2])
                cols2 = jnp.concatenate(pieces, axis=0)
                y2 = _dot(w2_ref[...], cols2) + b2_ref[...]
                pooled = y2 if pooled is None else jnp.maximum(pooled, y2)
        out_ref[:, j, :] = pooled.astype(jnp.bfloat16)


def _fc_stage_body(xp_ref, w3_ref, fc1b_ref, fc2w_ref, fc2b_ref, len_ref,
                   vals_ref, idx_ref, acc_ref):
    c = pl.program_id(0)

    @pl.when(c == 0)
    def _():
        acc_ref[...] = jnp.zeros_like(acc_ref)

    acc_ref[...] += _dot(xp_ref[0], w3_ref[0])

    @pl.when(c == pl.num_programs(0) - 1)
    def _():
        y = jnp.maximum(acc_ref[...] + fc1b_ref[...], 0.0)
        logits = _dot(y, fc2w_ref[...]) + fc2b_ref[...]
        mx = jnp.max(logits, axis=1, keepdims=True)
        e = jnp.exp(logits - mx)
        probs = e / jnp.sum(e, axis=1, keepdims=True)
        row_l = jax.lax.broadcasted_iota(jnp.int32, (448, 1), 0) % 7
        keep = (row_l < len_ref[...]).astype(jnp.float32)
        lanes = jax.lax.broadcasted_iota(jnp.int32, (448, 16), 1)
        vals_ref[...] = jnp.zeros_like(vals_ref)
        idx_ref[...] = jnp.zeros_like(idx_ref)
        work = probs
        for k in range(7):
            top = jnp.max(work, axis=1, keepdims=True)
            sel = jnp.min(jnp.where(work == top, lanes, 9999), axis=1,
                          keepdims=True)
            vals_ref[:, k:k + 1] = top * keep
            idx_ref[:, k:k + 1] = sel
            work = jnp.where(lanes == sel, -jnp.inf, work)


def kernel(img_seq, img_seq_len, conv1_w, conv1_b, conv2_w, conv2_b,
           fc1_w, fc1_b, fc2_w, fc2_b):
    B, L = img_seq.shape[0], img_seq.shape[1]
    N = B * L
    x = img_seq.reshape(N, 45, 45)
    quarters = []
    for pr in range(2):
        for pc in range(2):
            q = x[:, pr::2, pc::2]
            q = jnp.pad(q, ((0, 0), (1, 25 - q.shape[1]), (1, 23 - q.shape[2])))
            quarters.append(q.reshape(N, _NQ))
    xq = jnp.concatenate(quarters, axis=1)

    w1f = jnp.einsum('ct,jk->jctk', conv1_w.reshape(32, 9),
                     jnp.eye(_NB, dtype=jnp.float32)).reshape(_NB * 32, _NB * 9)
    b1 = jnp.tile(conv1_b.reshape(32, 1), (_NB, 1))
    w2r = conv2_w.transpose(0, 2, 3, 1).reshape(64, 288)
    b2 = conv2_b.reshape(64, 1)

    pooled = pl.pallas_call(
        _conv_stage_body,
        grid=(N // _NB,),
        in_specs=[
            pl.BlockSpec((_NB, 4 * _NQ), lambda i: (i, 0)),
            pl.BlockSpec((_NB * 32, _NB * 9), lambda i: (0, 0)),
            pl.BlockSpec((_NB * 32, 1), lambda i: (0, 0)),
            pl.BlockSpec((64, 288), lambda i: (0, 0)),
            pl.BlockSpec((64, 1), lambda i: (0, 0)),
        ],
        out_specs=pl.BlockSpec((64, _NB, _NY2), lambda i: (0, i, 0)),
        out_shape=jax.ShapeDtypeStruct((64, N, _NY2), jnp.bfloat16),
    )(xq, w1f, b1, w2r, b2)

    w3 = fc1_w.reshape(128, 64, 22, 22)
    w3 = jnp.pad(w3, ((0, 0), (0, 0), (0, 0), (0, 2)))
    w3 = w3.reshape(128, 64, _NY2).transpose(1, 2, 0).astype(jnp.bfloat16)
    fc1b = fc1_b.reshape(1, 128)
    fc2wt = jnp.zeros((128, 16), jnp.float32).at[:, :14].set(fc2_w.T)
    fc2b = jnp.full((1, 16), -1e30, jnp.float32).at[0, :14].set(fc2_b)
    len_rep = jnp.repeat(img_seq_len.astype(jnp.int32), L).reshape(N, 1)

    vals, idx = pl.pallas_call(
        _fc_stage_body,
        grid=(64,),
        in_specs=[
            pl.BlockSpec((1, N, _NY2), lambda c: (c, 0, 0)),
            pl.BlockSpec((1, _NY2, 128), lambda c: (c, 0, 0)),
            pl.BlockSpec((1, 128), lambda c: (0, 0)),
            pl.BlockSpec((128, 16), lambda c: (0, 0)),
            pl.BlockSpec((1, 16), lambda c: (0, 0)),
            pl.BlockSpec((N, 1), lambda c: (0, 0)),
        ],
        out_specs=[
            pl.BlockSpec((N, 16), lambda c: (0, 0)),
            pl.BlockSpec((N, 16), lambda c: (0, 0)),
        ],
        out_shape=[
            jax.ShapeDtypeStruct((N, 16), jnp.float32),
            jax.ShapeDtypeStruct((N, 16), jnp.int32),
        ],
        scratch_shapes=[pltpu.VMEM((N, 128), jnp.float32)],
    )(pooled, w3, fc1b, fc2wt, fc2b, len_rep)

    top_vals = vals[:, :7].reshape(B, L, 7)
    top_idx = idx[:, :7].reshape(B, L, 7)
    return top_vals, top_idx


# speedup vs baseline: 1.5910x; 1.1701x over previous
import jax
import jax.numpy as jnp
from jax.experimental import pallas as pl
from jax.experimental.pallas import tpu as pltpu


def _dot(a, b):
    return jnp.dot(a.astype(jnp.bfloat16), b.astype(jnp.bfloat16),
                   preferred_element_type=jnp.float32)


_QW = 24
_NQ = 624
_NY1 = 552
_NY2 = 528
_NB = 16


def _tap_src(q, d):
    t = q + d - 1
    p = t % 2
    s = (t - p) // 2
    return p, s


def _conv_stage_body(xq_ref, w1_ref, b1_ref, w2_ref, b2_ref, out_ref):
    y1s = {}
    for qr in range(2):
        for qc in range(2):
            taps = []
            for di in range(3):
                pr, a = _tap_src(qr, di)
                for dj in range(3):
                    pc, b = _tap_src(qc, dj)
                    off = (2 * pr + pc) * _NQ + (1 + a) * _QW + (1 + b)
                    taps.append(xq_ref[:, off:off + _NY1])
            cols1 = jnp.concatenate(taps, axis=0)
            y1 = _dot(w1_ref[...], cols1) + b1_ref[...]
            y1 = jnp.maximum(y1, 0.0)
            if qc == 1:
                uv = jax.lax.broadcasted_iota(jnp.int32, (1, _NY1), 1)
                y1 = y1 * (uv % _QW < _QW - 1).astype(jnp.float32)
            y1 = y1.astype(jnp.bfloat16)
            y1m = jnp.concatenate(
                [jnp.zeros((_NB * 32, _QW + 1), jnp.bfloat16), y1,
                 jnp.zeros((_NB * 32, _NQ - _NY1 - _QW - 1), jnp.bfloat16)],
                axis=1)
            for ral in ((1, 2) if qr == 0 else (0, 1)):
                for rbe in ((1, 2) if qc == 0 else (0, 1)):
                    o = ral * _QW + rbe
                    y1s[(qr, qc, ral, rbe)] = y1m[:, o:o + _NY2]
    for j in range(_NB):
        pooled = None
        for a in range(2):
            for b in range(2):
                pieces = []
                for di in range(3):
                    pr, al = _tap_src(a, di)
                    for dj in range(3):
                        pc, be = _tap_src(b, dj)
                        pieces.append(
                            y1s[(pr, pc, 1 + al, 1 + be)][j * 32:(j + 1) * 32])
                cols2 = jnp.concatenate(pieces, axis=0)
                y2 = _dot(w2_ref[...], cols2)
                pooled = y2 if pooled is None else jnp.maximum(pooled, y2)
        out_ref[:, j, :] = (pooled + b2_ref[...]).astype(jnp.bfloat16)


def _fc_stage_body(xp_ref, w3_ref, fc1b_ref, fc2w_ref, fc2b_ref, len_ref,
                   vals_ref, idx_ref, acc_ref):
    c = pl.program_id(0)

    @pl.when(c == 0)
    def _():
        acc_ref[...] = jnp.zeros_like(acc_ref)

    acc_ref[...] += _dot(xp_ref[0], w3_ref[0])

    @pl.when(c == pl.num_programs(0) - 1)
    def _():
        y = jnp.maximum(acc_ref[...] + fc1b_ref[...], 0.0)
        logits = _dot(y, fc2w_ref[...]) + fc2b_ref[...]
        mx = jnp.max(logits, axis=1, keepdims=True)
        e = jnp.exp(logits - mx)
        probs = e / jnp.sum(e, axis=1, keepdims=True)
        row_l = jax.lax.broadcasted_iota(jnp.int32, (448, 1), 0) % 7
        keep = (row_l < len_ref[...]).astype(jnp.float32)
        lanes = jax.lax.broadcasted_iota(jnp.int32, (448, 16), 1)
        vals_ref[...] = jnp.zeros_like(vals_ref)
        idx_ref[...] = jnp.zeros_like(idx_ref)
        work = probs
        for k in range(7):
            top = jnp.max(work, axis=1, keepdims=True)
            sel = jnp.min(jnp.where(work == top, lanes, 9999), axis=1,
                          keepdims=True)
            vals_ref[:, k:k + 1] = top * keep
            idx_ref[:, k:k + 1] = sel
            work = jnp.where(lanes == sel, -jnp.inf, work)


def kernel(img_seq, img_seq_len, conv1_w, conv1_b, conv2_w, conv2_b,
           fc1_w, fc1_b, fc2_w, fc2_b):
    B, L = img_seq.shape[0], img_seq.shape[1]
    N = B * L
    x = img_seq.reshape(N, 45, 45)
    quarters = []
    for pr in range(2):
        for pc in range(2):
            q = x[:, pr::2, pc::2]
            q = jnp.pad(q, ((0, 0), (1, 25 - q.shape[1]), (1, 23 - q.shape[2])))
            quarters.append(q.reshape(N, _NQ))
    xq = jnp.concatenate(quarters, axis=1).astype(jnp.bfloat16)

    w1f = jnp.einsum('ct,jk->jctk', conv1_w.reshape(32, 9),
                     jnp.eye(_NB, dtype=jnp.float32))
    w1f = w1f.reshape(_NB * 32, _NB * 9).astype(jnp.bfloat16)
    b1 = jnp.tile(conv1_b.reshape(32, 1), (_NB, 1))
    w2r = conv2_w.transpose(0, 2, 3, 1).reshape(64, 288).astype(jnp.bfloat16)
    b2 = conv2_b.reshape(64, 1)

    pooled = pl.pallas_call(
        _conv_stage_body,
        grid=(N // _NB,),
        in_specs=[
            pl.BlockSpec((_NB, 4 * _NQ), lambda i: (i, 0)),
            pl.BlockSpec((_NB * 32, _NB * 9), lambda i: (0, 0)),
            pl.BlockSpec((_NB * 32, 1), lambda i: (0, 0)),
            pl.BlockSpec((64, 288), lambda i: (0, 0)),
            pl.BlockSpec((64, 1), lambda i: (0, 0)),
        ],
        out_specs=pl.BlockSpec((64, _NB, _NY2), lambda i: (0, i, 0)),
        out_shape=jax.ShapeDtypeStruct((64, N, _NY2), jnp.bfloat16),
    )(xq, w1f, b1, w2r, b2)

    w3 = fc1_w.reshape(128, 64, 22, 22)
    w3 = jnp.pad(w3, ((0, 0), (0, 0), (0, 0), (0, 2)))
    w3 = w3.reshape(128, 64, _NY2).transpose(1, 2, 0).astype(jnp.bfloat16)
    fc1b = fc1_b.reshape(1, 128)
    fc2wt = jnp.zeros((128, 16), jnp.float32).at[:, :14].set(fc2_w.T)
    fc2b = jnp.full((1, 16), -1e30, jnp.float32).at[0, :14].set(fc2_b)
    len_rep = jnp.repeat(img_seq_len.astype(jnp.int32), L).reshape(N, 1)

    vals, idx = pl.pallas_call(
        _fc_stage_body,
        grid=(64,),
        in_specs=[
            pl.BlockSpec((1, N, _NY2), lambda c: (c, 0, 0)),
            pl.BlockSpec((1, _NY2, 128), lambda c: (c, 0, 0)),
            pl.BlockSpec((1, 128), lambda c: (0, 0)),
            pl.BlockSpec((128, 16), lambda c: (0, 0)),
            pl.BlockSpec((1, 16), lambda c: (0, 0)),
            pl.BlockSpec((N, 1), lambda c: (0, 0)),
        ],
        out_specs=[
            pl.BlockSpec((N, 16), lambda c: (0, 0)),
            pl.BlockSpec((N, 16), lambda c: (0, 0)),
        ],
        out_shape=[
            jax.ShapeDtypeStruct((N, 16), jnp.float32),
            jax.ShapeDtypeStruct((N, 16), jnp.int32),
        ],
        scratch_shapes=[pltpu.VMEM((N, 128), jnp.float32)],
    )(pooled, w3, fc1b, fc2wt, fc2b, len_rep)

    top_vals = vals[:, :7].reshape(B, L, 7)
    top_idx = idx[:, :7].reshape(B, L, 7)
    return top_vals, top_idx


# speedup vs baseline: 1.6640x; 1.0459x over previous
import jax
import jax.numpy as jnp
from jax.experimental import pallas as pl
from jax.experimental.pallas import tpu as pltpu


def _dot(a, b):
    return jnp.dot(a.astype(jnp.bfloat16), b.astype(jnp.bfloat16),
                   preferred_element_type=jnp.float32)


_QW = 24
_NQ = 624
_NY1 = 552
_NY2 = 528
_NB = 32


def _tap_src(q, d):
    t = q + d - 1
    p = t % 2
    s = (t - p) // 2
    return p, s


def _conv_stage_body(xq_ref, w1_ref, b1_ref, w2_ref, b2_ref, out_ref):
    y1s = {}
    for qr in range(2):
        for qc in range(2):
            taps = []
            for di in range(3):
                pr, a = _tap_src(qr, di)
                for dj in range(3):
                    pc, b = _tap_src(qc, dj)
                    off = (2 * pr + pc) * _NQ + (1 + a) * _QW + (1 + b)
                    taps.append(xq_ref[:, off:off + _NY1])
            cols1 = jnp.concatenate(taps, axis=0)
            y1 = _dot(w1_ref[...], cols1) + b1_ref[...]
            y1 = jnp.maximum(y1, 0.0)
            if qc == 1:
                uv = jax.lax.broadcasted_iota(jnp.int32, (1, _NY1), 1)
                y1 = y1 * (uv % _QW < _QW - 1).astype(jnp.float32)
            y1 = y1.astype(jnp.bfloat16)
            y1m = jnp.concatenate(
                [jnp.zeros((_NB * 32, _QW + 1), jnp.bfloat16), y1,
                 jnp.zeros((_NB * 32, _NQ - _NY1 - _QW - 1), jnp.bfloat16)],
                axis=1)
            for ral in ((1, 2) if qr == 0 else (0, 1)):
                for rbe in ((1, 2) if qc == 0 else (0, 1)):
                    o = ral * _QW + rbe
                    y1s[(qr, qc, ral, rbe)] = y1m[:, o:o + _NY2]
    for j in range(_NB):
        pooled = None
        for a in range(2):
            for b in range(2):
                pieces = []
                for di in range(3):
                    pr, al = _tap_src(a, di)
                    for dj in range(3):
                        pc, be = _tap_src(b, dj)
                        pieces.append(
                            y1s[(pr, pc, 1 + al, 1 + be)][j * 32:(j + 1) * 32])
                cols2 = jnp.concatenate(pieces, axis=0)
                y2 = _dot(w2_ref[...], cols2)
                pooled = y2 if pooled is None else jnp.maximum(pooled, y2)
        out_ref[:, j, :] = (pooled + b2_ref[...]).astype(jnp.bfloat16)


def _fc_stage_body(xp_ref, w3_ref, fc1b_ref, fc2w_ref, fc2b_ref, len_ref,
                   vals_ref, idx_ref, acc_ref):
    c = pl.program_id(0)

    @pl.when(c == 0)
    def _():
        acc_ref[...] = jnp.zeros_like(acc_ref)

    acc_ref[...] += _dot(xp_ref[0], w3_ref[0])

    @pl.when(c == pl.num_programs(0) - 1)
    def _():
        y = jnp.maximum(acc_ref[...] + fc1b_ref[...], 0.0)
        logits = _dot(y, fc2w_ref[...]) + fc2b_ref[...]
        mx = jnp.max(logits, axis=1, keepdims=True)
        e = jnp.exp(logits - mx)
        probs = e / jnp.sum(e, axis=1, keepdims=True)
        row_l = jax.lax.broadcasted_iota(jnp.int32, (448, 1), 0) % 7
        keep = (row_l < len_ref[...]).astype(jnp.float32)
        lanes = jax.lax.broadcasted_iota(jnp.int32, (448, 16), 1)
        vals_ref[...] = jnp.zeros_like(vals_ref)
        idx_ref[...] = jnp.zeros_like(idx_ref)
        work = probs
        for k in range(7):
            top = jnp.max(work, axis=1, keepdims=True)
            sel = jnp.min(jnp.where(work == top, lanes, 9999), axis=1,
                          keepdims=True)
            vals_ref[:, k:k + 1] = top * keep
            idx_ref[:, k:k + 1] = sel
            work = jnp.where(lanes == sel, -jnp.inf, work)


def kernel(img_seq, img_seq_len, conv1_w, conv1_b, conv2_w, conv2_b,
           fc1_w, fc1_b, fc2_w, fc2_b):
    B, L = img_seq.shape[0], img_seq.shape[1]
    N = B * L
    x = img_seq.reshape(N, 45, 45)
    quarters = []
    for pr in range(2):
        for pc in range(2):
            q = x[:, pr::2, pc::2]
            q = jnp.pad(q, ((0, 0), (1, 25 - q.shape[1]), (1, 23 - q.shape[2])))
            quarters.append(q.reshape(N, _NQ))
    xq = jnp.concatenate(quarters, axis=1).astype(jnp.bfloat16)

    w1f = jnp.einsum('ct,jk->jctk', conv1_w.reshape(32, 9),
                     jnp.eye(_NB, dtype=jnp.float32))
    w1f = w1f.reshape(_NB * 32, _NB * 9).astype(jnp.bfloat16)
    b1 = jnp.tile(conv1_b.reshape(32, 1), (_NB, 1))
    w2r = conv2_w.transpose(0, 2, 3, 1).reshape(64, 288).astype(jnp.bfloat16)
    b2 = conv2_b.reshape(64, 1)

    pooled = pl.pallas_call(
        _conv_stage_body,
        grid=(N // _NB,),
        in_specs=[
            pl.BlockSpec((_NB, 4 * _NQ), lambda i: (i, 0)),
            pl.BlockSpec((_NB * 32, _NB * 9), lambda i: (0, 0)),
            pl.BlockSpec((_NB * 32, 1), lambda i: (0, 0)),
            pl.BlockSpec((64, 288), lambda i: (0, 0)),
            pl.BlockSpec((64, 1), lambda i: (0, 0)),
        ],
        out_specs=pl.BlockSpec((64, _NB, _NY2), lambda i: (0, i, 0)),
        out_shape=jax.ShapeDtypeStruct((64, N, _NY2), jnp.bfloat16),
    )(xq, w1f, b1, w2r, b2)

    w3 = fc1_w.reshape(128, 64, 22, 22)
    w3 = jnp.pad(w3, ((0, 0), (0, 0), (0, 0), (0, 2)))
    w3 = w3.reshape(128, 64, _NY2).transpose(1, 2, 0).astype(jnp.bfloat16)
    fc1b = fc1_b.reshape(1, 128)
    fc2wt = jnp.zeros((128, 16), jnp.float32).at[:, :14].set(fc2_w.T)
    fc2b = jnp.full((1, 16), -1e30, jnp.float32).at[0, :14].set(fc2_b)
    len_rep = jnp.repeat(img_seq_len.astype(jnp.int32), L).reshape(N, 1)

    vals, idx = pl.pallas_call(
        _fc_stage_body,
        grid=(64,),
        in_specs=[
            pl.BlockSpec((1, N, _NY2), lambda c: (c, 0, 0)),
            pl.BlockSpec((1, _NY2, 128), lambda c: (c, 0, 0)),
            pl.BlockSpec((1, 128), lambda c: (0, 0)),
            pl.BlockSpec((128, 16), lambda c: (0, 0)),
            pl.BlockSpec((1, 16), lambda c: (0, 0)),
            pl.BlockSpec((N, 1), lambda c: (0, 0)),
        ],
        out_specs=[
            pl.BlockSpec((N, 16), lambda c: (0, 0)),
            pl.BlockSpec((N, 16), lambda c: (0, 0)),
        ],
        out_shape=[
            jax.ShapeDtypeStruct((N, 16), jnp.float32),
            jax.ShapeDtypeStruct((N, 16), jnp.int32),
        ],
        scratch_shapes=[pltpu.VMEM((N, 128), jnp.float32)],
    )(pooled, w3, fc1b, fc2wt, fc2b, len_rep)

    top_vals = vals[:, :7].reshape(B, L, 7)
    top_idx = idx[:, :7].reshape(B, L, 7)
    return top_vals, top_idx
